# baseline (speedup 1.0000x reference)
import functools

import jax
import jax.numpy as jnp
from jax import lax
from jax.experimental import pallas as pl
from jax.experimental.pallas import tpu as pltpu
from jax.experimental.pallas import tpu_sc as plsc

D_MODEL = 128
NC, NS = 2, 16
NW = NC * NS
CHUNK = 128
W = 2
B_TOT = 4096 * 200
CH_PER_W = B_TOT // (NW * CHUNK)
NBUF = 3
PAIRS = CH_PER_W // W
GROUPS = PAIRS // NBUF
REM = PAIRS - GROUPS * NBUF

_mesh = plsc.VectorSubcoreMesh(core_axis_name="c", subcore_axis_name="s")


@functools.partial(
    pl.kernel,
    mesh=_mesh,
    out_type=jax.ShapeDtypeStruct((B_TOT // CHUNK, CHUNK, D_MODEL), jnp.float32),
    scratch_types=[
        pltpu.VMEM((CH_PER_W, CHUNK), jnp.int32),
        pltpu.VMEM_SHARED((24, D_MODEL), jnp.float32),
        pltpu.VMEM((W, CHUNK, D_MODEL), jnp.float32),
        pltpu.VMEM((W, CHUNK, D_MODEL), jnp.float32),
        pltpu.VMEM((W, CHUNK, D_MODEL), jnp.float32),
        pltpu.SemaphoreType.DMA,
        pltpu.SemaphoreType.DMA,
        pltpu.SemaphoreType.DMA,
        pltpu.SemaphoreType.DMA,
        pltpu.SemaphoreType.DMA,
        pltpu.SemaphoreType.DMA,
    ],
)
def _gather_kernel(idx_hbm, table_hbm, out_hbm, idx_v, table_sh,
                   r0, r1, r2, g0, g1, g2, o0, o1, o2):
    rows = (r0, r1, r2)
    sem_g = (g0, g1, g2)
    sem_o = (o0, o1, o2)
    sid = lax.axis_index("s")
    wid = sid * NC + lax.axis_index("c")
    base = wid * CH_PER_W

    @pl.when(sid == 0)
    def _():
        pltpu.sync_copy(table_hbm, table_sh)

    pltpu.sync_copy(idx_hbm.at[wid], idx_v)
    plsc.subcore_barrier()

    def body(g, carry):
        p0 = g * NBUF
        descs = []
        for b in range(NBUF):
            @pl.when(g > 0)
            def _(b=b):
                pltpu.make_async_copy(
                    rows[b], out_hbm.at[pl.ds(base, W)], sem_o[b]).wait()
            for k in range(W):
                descs.append(pltpu.async_copy(
                    table_sh.at[idx_v.at[(p0 + b) * W + k]],
                    rows[b].at[k], sem_g[b]))
        for b in range(NBUF):
            descs[2 * b].wait()
            descs[2 * b + 1].wait()
            pltpu.async_copy(
                rows[b], out_hbm.at[pl.ds(base + (p0 + b) * W, W)], sem_o[b])
        return carry

    lax.fori_loop(0, GROUPS, body, 0)

    pltpu.make_async_copy(rows[0], out_hbm.at[pl.ds(base, W)], sem_o[0]).wait()
    last = []
    for k in range(W):
        last.append(pltpu.async_copy(
            table_sh.at[idx_v.at[(PAIRS - 1) * W + k]], rows[0].at[k], sem_g[0]))
    for d in last:
        d.wait()
    pltpu.async_copy(
        rows[0], out_hbm.at[pl.ds(base + (PAIRS - 1) * W, W)], sem_o[0])

    pltpu.make_async_copy(rows[0], out_hbm.at[pl.ds(base, W)], sem_o[0]).wait()
    for b in range(1, NBUF):
        pltpu.make_async_copy(
            rows[b], out_hbm.at[pl.ds(base, W)], sem_o[b]).wait()


def kernel(x, pe):
    idx = x.reshape(NW, CH_PER_W, CHUNK)
    out = _gather_kernel(idx, pe)
    return out.reshape(x.shape[0], x.shape[1], D_MODEL)


# speedup vs baseline: 1.0371x; 1.0371x over previous
import functools

import jax
import jax.numpy as jnp
from jax import lax
from jax.experimental import pallas as pl
from jax.experimental.pallas import tpu as pltpu
from jax.experimental.pallas import tpu_sc as plsc

D_MODEL = 128
NC, NS = 2, 16
NW = NC * NS
CHUNK = 128
B_TOT = 4096 * 200
CH_PER_W = B_TOT // (NW * CHUNK)
NBUF = 4
GROUPS = CH_PER_W // NBUF

_mesh = plsc.VectorSubcoreMesh(core_axis_name="c", subcore_axis_name="s")


@functools.partial(
    pl.kernel,
    mesh=_mesh,
    out_type=jax.ShapeDtypeStruct((B_TOT, D_MODEL), jnp.float32),
    scratch_types=[
        pltpu.VMEM((CH_PER_W, CHUNK), jnp.int32),
        pltpu.VMEM_SHARED((24, D_MODEL), jnp.float32),
        pltpu.VMEM((NBUF, CHUNK, D_MODEL), jnp.float32),
        pltpu.SemaphoreType.DMA,
        pltpu.SemaphoreType.DMA,
        pltpu.SemaphoreType.DMA,
        pltpu.SemaphoreType.DMA,
        pltpu.SemaphoreType.DMA,
        pltpu.SemaphoreType.DMA,
        pltpu.SemaphoreType.DMA,
        pltpu.SemaphoreType.DMA,
    ],
)
def _gather_kernel(idx_hbm, table_hbm, out_hbm, idx_v, table_sh, rows_v,
                   g0, g1, g2, g3, o0, o1, o2, o3):
    sem_g = (g0, g1, g2, g3)
    sem_o = (o0, o1, o2, o3)
    sid = lax.axis_index("s")
    wid = sid * NC + lax.axis_index("c")
    base = wid * (CH_PER_W * CHUNK)

    @pl.when(sid == 0)
    def _():
        pltpu.sync_copy(table_hbm, table_sh)

    pltpu.sync_copy(idx_hbm.at[wid], idx_v)
    plsc.subcore_barrier()

    def body(g, carry):
        j0 = g * NBUF
        descs = []
        for b in range(NBUF):
            @pl.when(g > 0)
            def _(b=b, j0=j0):
                pltpu.make_async_copy(
                    rows_v.at[b],
                    out_hbm.at[pl.ds(base + (j0 + b - NBUF) * CHUNK, CHUNK)],
                    sem_o[b]).wait()
            descs.append(pltpu.async_copy(
                table_sh.at[idx_v.at[j0 + b]], rows_v.at[b], sem_g[b]))
        for b in range(NBUF):
            descs[b].wait()
            pltpu.async_copy(
                rows_v.at[b],
                out_hbm.at[pl.ds(base + (j0 + b) * CHUNK, CHUNK)],
                sem_o[b])
        return carry

    lax.fori_loop(0, GROUPS, body, 0)
    for b in range(NBUF):
        pltpu.make_async_copy(
            rows_v.at[b],
            out_hbm.at[pl.ds(base + b * CHUNK, CHUNK)],
            sem_o[b]).wait()


def kernel(x, pe):
    idx = x.reshape(NW, CH_PER_W, CHUNK)
    out = _gather_kernel(idx, pe)
    return out.reshape(x.shape[0], x.shape[1], D_MODEL)


# speedup vs baseline: 1.0401x; 1.0028x over previous
import functools

import jax
import jax.numpy as jnp
from jax import lax
from jax.experimental import pallas as pl
from jax.experimental.pallas import tpu as pltpu
from jax.experimental.pallas import tpu_sc as plsc

D_MODEL = 128
NC, NS = 2, 16
NW = NC * NS
CHUNK = 64
B_TOT = 4096 * 200
CH_PER_W = B_TOT // (NW * CHUNK)
NBUF = 8
GROUPS = CH_PER_W // NBUF

_mesh = plsc.VectorSubcoreMesh(core_axis_name="c", subcore_axis_name="s")


@functools.partial(
    pl.kernel,
    mesh=_mesh,
    out_type=jax.ShapeDtypeStruct((B_TOT, D_MODEL), jnp.float32),
    scratch_types=[
        pltpu.VMEM((CH_PER_W, CHUNK), jnp.int32),
        pltpu.VMEM_SHARED((24, D_MODEL), jnp.float32),
        pltpu.VMEM((NBUF, CHUNK, D_MODEL), jnp.float32),
        pltpu.SemaphoreType.DMA,
        pltpu.SemaphoreType.DMA,
        pltpu.SemaphoreType.DMA,
        pltpu.SemaphoreType.DMA,
        pltpu.SemaphoreType.DMA,
        pltpu.SemaphoreType.DMA,
        pltpu.SemaphoreType.DMA,
        pltpu.SemaphoreType.DMA,
        pltpu.SemaphoreType.DMA,
        pltpu.SemaphoreType.DMA,
        pltpu.SemaphoreType.DMA,
        pltpu.SemaphoreType.DMA,
        pltpu.SemaphoreType.DMA,
        pltpu.SemaphoreType.DMA,
        pltpu.SemaphoreType.DMA,
        pltpu.SemaphoreType.DMA,
    ],
)
def _gather_kernel(idx_hbm, table_hbm, out_hbm, idx_v, table_sh, rows_v,
                   g0, g1, g2, g3, g4, g5, g6, g7,
                   o0, o1, o2, o3, o4, o5, o6, o7):
    sem_g = (g0, g1, g2, g3, g4, g5, g6, g7)
    sem_o = (o0, o1, o2, o3, o4, o5, o6, o7)
    sid = lax.axis_index("s")
    wid = sid * NC + lax.axis_index("c")
    base = wid * (CH_PER_W * CHUNK)

    @pl.when(sid == 0)
    def _():
        pltpu.sync_copy(table_hbm, table_sh)

    pltpu.sync_copy(idx_hbm.at[wid], idx_v)
    plsc.subcore_barrier()

    def body(g, carry):
        j0 = g * NBUF
        descs = []
        for b in range(NBUF):
            @pl.when(g > 0)
            def _(b=b, j0=j0):
                pltpu.make_async_copy(
                    rows_v.at[b],
                    out_hbm.at[pl.ds(base + (j0 + b - NBUF) * CHUNK, CHUNK)],
                    sem_o[b]).wait()
            descs.append(pltpu.async_copy(
                table_sh.at[idx_v.at[j0 + b]], rows_v.at[b], sem_g[b]))
        for b in range(NBUF):
            descs[b].wait()
            pltpu.async_copy(
                rows_v.at[b],
                out_hbm.at[pl.ds(base + (j0 + b) * CHUNK, CHUNK)],
                sem_o[b])
        return carry

    lax.fori_loop(0, GROUPS, body, 0)
    for b in range(NBUF):
        pltpu.make_async_copy(
            rows_v.at[b],
            out_hbm.at[pl.ds(base + b * CHUNK, CHUNK)],
            sem_o[b]).wait()


def kernel(x, pe):
    idx = x.reshape(NW, CH_PER_W, CHUNK)
    out = _gather_kernel(idx, pe)
    return out.reshape(x.shape[0], x.shape[1], D_MODEL)
